# Initial kernel scaffold; baseline (speedup 1.0000x reference)
import jax
import jax.numpy as jnp
from jax.experimental import pallas as pl


def kernel(x, indices, W0, b0, W1, b1, W2, b2, W3, b3, Wf1, bf1, Wf2, bf2):
    raise NotImplementedError("write your pallas kernel here")


# speedup vs baseline: 3.0597x; 3.0597x over previous
import functools

import jax
import jax.numpy as jnp
from jax import lax
from jax.experimental import pallas as pl
from jax.experimental.pallas import tpu as pltpu
from jax.experimental.pallas import tpu_sc as plsc

NNODES = 50000
L = 16
B = NNODES * L
NC, NS = 2, 16
NW = NC * NS
GBLK = 128
NB = B // GBLK
TPW = -(-NB // NW)


def _make_sc_gather(D):
  mesh = plsc.VectorSubcoreMesh(core_axis_name="c", subcore_axis_name="s")

  @functools.partial(
      pl.kernel,
      mesh=mesh,
      out_type=jax.ShapeDtypeStruct((B, D), jnp.float32),
      scratch_types=[
          pltpu.VMEM((GBLK,), jnp.int32),
          pltpu.VMEM((GBLK, D), jnp.float32),
          pltpu.SemaphoreType.DMA,
      ],
      compiler_params=pltpu.CompilerParams(use_tc_tiling_on_sc=False),
  )
  def gather_k(table_hbm, idx_hbm, out_hbm, idx_v, rows_v, sem):
    wid = lax.axis_index("s") * NC + lax.axis_index("c")

    def body(t, carry):
      blk = t * NW + wid

      @pl.when(blk < NB)
      def _():
        off = blk * GBLK
        pltpu.sync_copy(idx_hbm.at[pl.ds(off, GBLK)], idx_v)
        pltpu.async_copy(table_hbm.at[idx_v], rows_v, sem).wait()
        pltpu.sync_copy(rows_v, out_hbm.at[pl.ds(off, GBLK)])

      return carry

    lax.fori_loop(0, TPW, body, 0)

  return gather_k


_gather = {D: _make_sc_gather(D) for D in (16, 32, 64)}


def _elu(v):
  return jnp.where(v > 0, v, jnp.exp(v) - 1.0)


def _mm_call(g, w, b, elu, rows):
  k = g.shape[1]
  cout = w.shape[0]

  def mm_k(g_ref, w_ref, b_ref, o_ref):
    acc = lax.dot_general(g_ref[...], w_ref[...], (((1,), (1,)), ((), ())),
                          preferred_element_type=jnp.float32)
    acc = acc + b_ref[...]
    o_ref[...] = _elu(acc) if elu else acc

  return pl.pallas_call(
      mm_k,
      grid=(NNODES // rows,),
      in_specs=[
          pl.BlockSpec((rows, k), lambda i: (i, 0)),
          pl.BlockSpec((cout, k), lambda i: (0, 0)),
          pl.BlockSpec((1, cout), lambda i: (0, 0)),
      ],
      out_specs=pl.BlockSpec((rows, cout), lambda i: (i, 0)),
      out_shape=jax.ShapeDtypeStruct((NNODES, cout), jnp.float32),
  )(g, w, b)


def _final_call(g3, w3, b3, wc, bc, rows):
  k = g3.shape[1]
  cmid = w3.shape[0]
  cout = wc.shape[0]

  def fin_k(g_ref, w3_ref, b3_ref, wc_ref, bc_ref, o_ref):
    h = lax.dot_general(g_ref[...], w3_ref[...], (((1,), (1,)), ((), ())),
                        preferred_element_type=jnp.float32)
    h = _elu(h + b3_ref[...])
    o = lax.dot_general(h, wc_ref[...], (((1,), (1,)), ((), ())),
                        preferred_element_type=jnp.float32)
    o_ref[...] = o + bc_ref[...]

  return pl.pallas_call(
      fin_k,
      grid=(NNODES // rows,),
      in_specs=[
          pl.BlockSpec((rows, k), lambda i: (i, 0)),
          pl.BlockSpec((cmid, k), lambda i: (0, 0)),
          pl.BlockSpec((1, cmid), lambda i: (0, 0)),
          pl.BlockSpec((cout, cmid), lambda i: (0, 0)),
          pl.BlockSpec((1, cout), lambda i: (0, 0)),
      ],
      out_specs=pl.BlockSpec((rows, cout), lambda i: (i, 0)),
      out_shape=jax.ShapeDtypeStruct((NNODES, cout), jnp.float32),
  )(g3, w3, b3, wc, bc)


def kernel(x, indices, W0, b0, W1, b1, W2, b2, W3, b3, Wf1, bf1, Wf2, bf2):
  idx = indices.reshape(-1)

  h0 = _mm_call(x, W0, b0.reshape(1, -1), True, 2000)
  g1 = _gather[16](h0, idx).reshape(NNODES, 16 * L)
  h1 = _mm_call(g1, W1, b1.reshape(1, -1), True, 2000)
  g2 = _gather[32](h1, idx).reshape(NNODES, 32 * L)
  h2 = _mm_call(g2, W2, b2.reshape(1, -1), True, 2000)
  g3 = _gather[64](h2, idx).reshape(NNODES, 64 * L)

  wc = Wf2 @ Wf1
  bc = Wf2 @ bf1 + bf2
  return _final_call(g3, W3, b3.reshape(1, -1), wc, bc.reshape(1, -1), 2000)


# speedup vs baseline: 5.1657x; 1.6883x over previous
import functools

import jax
import jax.numpy as jnp
from jax import lax
from jax.experimental import pallas as pl
from jax.experimental.pallas import tpu as pltpu
from jax.experimental.pallas import tpu_sc as plsc

NNODES = 50000
L = 16
B = NNODES * L
NC, NS = 2, 16
NW = NC * NS
GBLK = 128
NB = B // GBLK
TPW = -(-NB // NW)


def _make_sc_gather(D, gsub):
  mesh = plsc.VectorSubcoreMesh(core_axis_name="c", subcore_axis_name="s")
  ch = gsub * GBLK
  nch = B // ch
  assert nch * ch == B
  spw = -(-nch // NW)
  assert spw % 2 == 0

  @functools.partial(
      pl.kernel,
      mesh=mesh,
      out_type=jax.ShapeDtypeStruct((B, D), jnp.float32),
      scratch_types=[
          pltpu.VMEM((2, gsub, GBLK), jnp.int32),
          pltpu.VMEM((2, ch, D), jnp.float32),
          pltpu.SemaphoreType.DMA,
          pltpu.SemaphoreType.DMA,
          pltpu.SemaphoreType.DMA,
          pltpu.SemaphoreType.DMA,
      ],
      compiler_params=pltpu.CompilerParams(use_tc_tiling_on_sc=False),
  )
  def gather_k(table_hbm, idx_hbm, out_hbm, idx_v, rows_v, g0, g1, s0, s1):
    wid = lax.axis_index("s") * NC + lax.axis_index("c")
    gsem = (g0, g1)
    ssem = (s0, s1)

    def fire(s, p):
      c = s * NW + wid

      @pl.when(c < nch)
      def _():
        off = c * ch
        pltpu.sync_copy(idx_hbm.at[pl.ds(c * gsub, gsub)], idx_v.at[p])
        for j in range(gsub):
          pltpu.async_copy(table_hbm.at[idx_v.at[p, j]],
                           rows_v.at[p, pl.ds(j * GBLK, GBLK)], gsem[p])

    def drain_and_store(s, p):
      c = s * NW + wid

      @pl.when(c < nch)
      def _():
        off = c * ch
        pltpu.make_async_copy(out_hbm.at[pl.ds(0, ch)], rows_v.at[p],
                              gsem[p]).wait()
        pltpu.async_copy(rows_v.at[p], out_hbm.at[pl.ds(off, ch)], ssem[p])

    def wait_store(s, p):
      c = s * NW + wid

      @pl.when(c < nch)
      def _():
        pltpu.make_async_copy(rows_v.at[p], out_hbm.at[pl.ds(0, ch)],
                              ssem[p]).wait()

    fire(0, 0)

    def body(s2, carry):
      for p in (0, 1):
        s = s2 * 2 + p
        q = 1 - p
        @pl.when(s >= 1)
        def _():
          wait_store(s - 1, q)

        fire(s + 1, q)
        drain_and_store(s, p)
      return carry

    lax.fori_loop(0, spw // 2, body, 0)
    wait_store(spw - 1, 1)

  return gather_k


_gather = {16: _make_sc_gather(16, 10),
           32: _make_sc_gather(32, 10),
           64: _make_sc_gather(64, 5)}


def _elu(v):
  return jnp.where(v > 0, v, jnp.exp(v) - 1.0)


def _mm_call(g, w, b, elu, rows):
  k = g.shape[1]
  cout = w.shape[0]

  def mm_k(g_ref, w_ref, b_ref, o_ref):
    acc = lax.dot_general(g_ref[...], w_ref[...], (((1,), (1,)), ((), ())),
                          preferred_element_type=jnp.float32)
    acc = acc + b_ref[...]
    o_ref[...] = _elu(acc) if elu else acc

  return pl.pallas_call(
      mm_k,
      grid=(NNODES // rows,),
      in_specs=[
          pl.BlockSpec((rows, k), lambda i: (i, 0)),
          pl.BlockSpec((cout, k), lambda i: (0, 0)),
          pl.BlockSpec((1, cout), lambda i: (0, 0)),
      ],
      out_specs=pl.BlockSpec((rows, cout), lambda i: (i, 0)),
      out_shape=jax.ShapeDtypeStruct((NNODES, cout), jnp.float32),
  )(g, w, b)


def _final_call(g3, w3, b3, wc, bc, rows):
  k = g3.shape[1]
  cmid = w3.shape[0]
  cout = wc.shape[0]

  def fin_k(g_ref, w3_ref, b3_ref, wc_ref, bc_ref, o_ref):
    h = lax.dot_general(g_ref[...], w3_ref[...], (((1,), (1,)), ((), ())),
                        preferred_element_type=jnp.float32)
    h = _elu(h + b3_ref[...])
    o = lax.dot_general(h, wc_ref[...], (((1,), (1,)), ((), ())),
                        preferred_element_type=jnp.float32)
    o_ref[...] = o + bc_ref[...]

  return pl.pallas_call(
      fin_k,
      grid=(NNODES // rows,),
      in_specs=[
          pl.BlockSpec((rows, k), lambda i: (i, 0)),
          pl.BlockSpec((cmid, k), lambda i: (0, 0)),
          pl.BlockSpec((1, cmid), lambda i: (0, 0)),
          pl.BlockSpec((cout, cmid), lambda i: (0, 0)),
          pl.BlockSpec((1, cout), lambda i: (0, 0)),
      ],
      out_specs=pl.BlockSpec((rows, cout), lambda i: (i, 0)),
      out_shape=jax.ShapeDtypeStruct((NNODES, cout), jnp.float32),
  )(g3, w3, b3, wc, bc)


def kernel(x, indices, W0, b0, W1, b1, W2, b2, W3, b3, Wf1, bf1, Wf2, bf2):
  idx = indices.reshape(NB, GBLK)

  h0 = _mm_call(x, W0, b0.reshape(1, -1), True, 2000)
  g1 = _gather[16](h0, idx).reshape(NNODES, 16 * L)
  h1 = _mm_call(g1, W1, b1.reshape(1, -1), True, 2000)
  g2 = _gather[32](h1, idx).reshape(NNODES, 32 * L)
  h2 = _mm_call(g2, W2, b2.reshape(1, -1), True, 2000)
  g3 = _gather[64](h2, idx).reshape(NNODES, 64 * L)

  wc = Wf2 @ Wf1
  bc = Wf2 @ bf1 + bf2
  return _final_call(g3, W3, b3.reshape(1, -1), wc, bc.reshape(1, -1), 2000)
